# Initial kernel scaffold; baseline (speedup 1.0000x reference)
import jax
import jax.numpy as jnp
from jax import lax
from jax.experimental import pallas as pl
from jax.experimental.pallas import tpu as pltpu
from jax.experimental.pallas import tpu_sc as plsc

B = 16384
L = 20
V = 1000000
D = 32

NC = 2
NS = 16
NW = NC * NS

TOTAL_BAGS = 2 * B
BAGS_PER_WORKER = TOTAL_BAGS // NW
CHUNK = 64
CHUNK_IDS = CHUNK * L
IDS_PER_GATHER = 128
GATHERS = CHUNK_IDS // IDS_PER_GATHER
CHUNKS = BAGS_PER_WORKER // CHUNK
IDX_ROWS_PER_CHUNK = CHUNK_IDS // IDS_PER_GATHER


def _sc_pool_kernel(ids_hbm, table_hbm, out_hbm, idx_v, rows_v, out_v, sem):
    wid = lax.axis_index("s") * NC + lax.axis_index("c")

    def chunk_body(i, carry):
        base_bag = wid * BAGS_PER_WORKER + i * CHUNK
        idx_row = base_bag * L // IDS_PER_GATHER
        pltpu.sync_copy(ids_hbm.at[pl.ds(idx_row, IDX_ROWS_PER_CHUNK)], idx_v)
        copies = []
        for j in range(GATHERS):
            copies.append(
                pltpu.async_copy(
                    table_hbm.at[idx_v.at[j]],
                    rows_v.at[pl.ds(j * IDS_PER_GATHER, IDS_PER_GATHER)],
                    sem,
                )
            )
        for c in copies:
            c.wait()

        def bag_body(b, carry2):
            r0 = b * L
            lo = rows_v[r0, pl.ds(0, 16)]
            hi = rows_v[r0, pl.ds(16, 16)]
            for k in range(1, L):
                lo = lo + rows_v[r0 + k, pl.ds(0, 16)]
                hi = hi + rows_v[r0 + k, pl.ds(16, 16)]
            out_v[b, pl.ds(0, 16)] = lo
            out_v[b, pl.ds(16, 16)] = hi
            return carry2

        lax.fori_loop(0, CHUNK, bag_body, 0)
        pltpu.sync_copy(out_v, out_hbm.at[pl.ds(base_bag, CHUNK)])
        return carry

    lax.fori_loop(0, CHUNKS, chunk_body, 0)


def _pooled(ids2d, table):
    mesh = plsc.VectorSubcoreMesh(
        core_axis_name="c", subcore_axis_name="s", num_cores=NC, num_subcores=NS
    )
    run = pl.kernel(
        _sc_pool_kernel,
        out_type=jax.ShapeDtypeStruct((TOTAL_BAGS, D), jnp.float32),
        mesh=mesh,
        scratch_types=[
            pltpu.VMEM((IDX_ROWS_PER_CHUNK, IDS_PER_GATHER), jnp.int32),
            pltpu.VMEM((CHUNK_IDS, D), jnp.float32),
            pltpu.VMEM((CHUNK, D), jnp.float32),
            pltpu.SemaphoreType.DMA,
        ],
    )
    return run(ids2d, table)


def kernel(values_f1, lengths_f1, values_f2, lengths_f2, table):
    del lengths_f1, lengths_f2
    ids = jnp.concatenate([values_f1, values_f2]).reshape(-1, IDS_PER_GATHER)
    pooled = _pooled(ids, table)
    return jnp.concatenate([pooled[:B], pooled[B:]], axis=1)


# speedup vs baseline: 13.2689x; 13.2689x over previous
import jax
import jax.numpy as jnp
from jax import lax
from jax.experimental import pallas as pl
from jax.experimental.pallas import tpu as pltpu
from jax.experimental.pallas import tpu_sc as plsc

B = 16384
L = 20
V = 1000000
D = 32

NC = 2
NS = 16
NW = NC * NS

TOTAL_BAGS = 2 * B
BAGS_PER_WORKER = TOTAL_BAGS // NW
CHUNK = 64
CHUNK_IDS = CHUNK * L
IDS_PER_GATHER = 128
GATHERS = CHUNK_IDS // IDS_PER_GATHER
CHUNKS = BAGS_PER_WORKER // CHUNK
IDX_ROWS_PER_CHUNK = CHUNK_IDS // IDS_PER_GATHER


def _sc_pool_kernel(ids_hbm, table_hbm, out_hbm, idx_v, rows_v, out_v, sem):
    wid = lax.axis_index("s") * NC + lax.axis_index("c")

    def chunk_body(i, carry):
        base_bag = wid * BAGS_PER_WORKER + i * CHUNK
        pltpu.sync_copy(ids_hbm.at[pl.ds(base_bag * L, CHUNK_IDS)], idx_v)
        copies = []
        for j in range(GATHERS):
            copies.append(
                pltpu.async_copy(
                    table_hbm.at[idx_v.at[pl.ds(j * IDS_PER_GATHER, IDS_PER_GATHER)]],
                    rows_v.at[pl.ds(j * IDS_PER_GATHER, IDS_PER_GATHER)],
                    sem,
                )
            )
        for c in copies:
            c.wait()

        def bag_body(b, carry2):
            r0 = b * L
            lo = rows_v[r0, pl.ds(0, 16)]
            hi = rows_v[r0, pl.ds(16, 16)]
            for k in range(1, L):
                lo = lo + rows_v[r0 + k, pl.ds(0, 16)]
                hi = hi + rows_v[r0 + k, pl.ds(16, 16)]
            out_v[b, pl.ds(0, 16)] = lo
            out_v[b, pl.ds(16, 16)] = hi
            return carry2

        lax.fori_loop(0, CHUNK, bag_body, 0)
        pltpu.sync_copy(out_v, out_hbm.at[pl.ds(base_bag, CHUNK)])
        return carry

    lax.fori_loop(0, CHUNKS, chunk_body, 0)


def _pooled(ids2d, table):
    mesh = plsc.VectorSubcoreMesh(
        core_axis_name="c", subcore_axis_name="s", num_cores=NC, num_subcores=NS
    )
    run = pl.kernel(
        _sc_pool_kernel,
        out_type=jax.ShapeDtypeStruct((TOTAL_BAGS, D), jnp.float32),
        mesh=mesh,
        scratch_types=[
            pltpu.VMEM((CHUNK_IDS,), jnp.int32),
            pltpu.VMEM((CHUNK_IDS, D), jnp.float32),
            pltpu.VMEM((CHUNK, D), jnp.float32),
            pltpu.SemaphoreType.DMA,
        ],
        compiler_params=pltpu.CompilerParams(use_tc_tiling_on_sc=False),
    )
    return run(ids2d, table)


def kernel(values_f1, lengths_f1, values_f2, lengths_f2, table):
    del lengths_f1, lengths_f2
    ids = jnp.concatenate([values_f1, values_f2])
    pooled = _pooled(ids, table)
    return jnp.concatenate([pooled[:B], pooled[B:]], axis=1)


# speedup vs baseline: 14.0909x; 1.0619x over previous
import jax
import jax.numpy as jnp
from jax import lax
from jax.experimental import pallas as pl
from jax.experimental.pallas import tpu as pltpu
from jax.experimental.pallas import tpu_sc as plsc

B = 16384
L = 20
V = 1000000
D = 32

NC = 2
NS = 16
NW = NC * NS

TOTAL_BAGS = 2 * B
BAGS_PER_WORKER = TOTAL_BAGS // NW
CHUNK = 64
CHUNK_IDS = CHUNK * L
IDS_PER_GATHER = 128
GATHERS = CHUNK_IDS // IDS_PER_GATHER
CHUNKS = BAGS_PER_WORKER // CHUNK
NBUF = 2


def _sc_pool_kernel(ids_hbm, table_hbm, out_hbm, idx_v, rows_v, out_v, gsem):
    wid = lax.axis_index("s") * NC + lax.axis_index("c")
    bag0 = wid * BAGS_PER_WORKER

    def fire(i, slot):
        base = (bag0 + i * CHUNK) * L
        pltpu.sync_copy(ids_hbm.at[pl.ds(base, CHUNK_IDS)], idx_v.at[slot])
        for j in range(GATHERS):
            pltpu.async_copy(
                table_hbm.at[idx_v.at[slot].at[pl.ds(j * IDS_PER_GATHER, IDS_PER_GATHER)]],
                rows_v.at[slot].at[pl.ds(j * IDS_PER_GATHER, IDS_PER_GATHER)],
                gsem.at[slot],
            )

    def drain(slot):
        for j in range(GATHERS):
            pltpu.make_async_copy(
                table_hbm.at[idx_v.at[slot].at[pl.ds(j * IDS_PER_GATHER, IDS_PER_GATHER)]],
                rows_v.at[slot].at[pl.ds(j * IDS_PER_GATHER, IDS_PER_GATHER)],
                gsem.at[slot],
            ).wait()

    def reduce_and_write(i, slot):
        rv = rows_v.at[slot]

        @plsc.parallel_loop(0, CHUNK, unroll=2)
        def _bag(b):
            r0 = b * L
            lo = rv[r0, pl.ds(0, 16)]
            hi = rv[r0, pl.ds(16, 16)]
            for k in range(1, L):
                lo = lo + rv[r0 + k, pl.ds(0, 16)]
                hi = hi + rv[r0 + k, pl.ds(16, 16)]
            out_v[b, pl.ds(0, 16)] = lo
            out_v[b, pl.ds(16, 16)] = hi

        pltpu.sync_copy(out_v, out_hbm.at[pl.ds(bag0 + i * CHUNK, CHUNK)])

    fire(0, 0)

    def pair_body(g, carry):
        for b in range(NBUF):
            i = 2 * g + b

            @pl.when(i + 1 < CHUNKS)
            def _():
                fire(i + 1, 1 - b)

            drain(b)
            reduce_and_write(i, b)
        return carry

    lax.fori_loop(0, CHUNKS // NBUF, pair_body, 0)


def _pooled(ids, table):
    mesh = plsc.VectorSubcoreMesh(
        core_axis_name="c", subcore_axis_name="s", num_cores=NC, num_subcores=NS
    )
    run = pl.kernel(
        _sc_pool_kernel,
        out_type=jax.ShapeDtypeStruct((TOTAL_BAGS, D), jnp.float32),
        mesh=mesh,
        scratch_types=[
            pltpu.VMEM((NBUF, CHUNK_IDS), jnp.int32),
            pltpu.VMEM((NBUF, CHUNK_IDS, D), jnp.float32),
            pltpu.VMEM((CHUNK, D), jnp.float32),
            pltpu.SemaphoreType.DMA((NBUF,)),
        ],
        compiler_params=pltpu.CompilerParams(use_tc_tiling_on_sc=False),
    )
    return run(ids, table)


def kernel(values_f1, lengths_f1, values_f2, lengths_f2, table):
    del lengths_f1, lengths_f2
    ids = jnp.concatenate([values_f1, values_f2])
    pooled = _pooled(ids, table)
    return jnp.concatenate([pooled[:B], pooled[B:]], axis=1)


# speedup vs baseline: 14.4890x; 1.0283x over previous
import jax
import jax.numpy as jnp
from jax import lax
from jax.experimental import pallas as pl
from jax.experimental.pallas import tpu as pltpu
from jax.experimental.pallas import tpu_sc as plsc

B = 16384
L = 20
V = 1000000
D = 32

NC = 2
NS = 16
NW = NC * NS

WORKERS_PER_FEATURE = NW // 2
BAGS_PER_WORKER = B // WORKERS_PER_FEATURE
CHUNK = 64
CHUNK_IDS = CHUNK * L
IDS_PER_GATHER = 128
GATHERS = CHUNK_IDS // IDS_PER_GATHER
CHUNKS = BAGS_PER_WORKER // CHUNK
NBUF = 2


def _sc_pool_kernel(v1_hbm, v2_hbm, table_hbm, out_hbm, idx_v, rows_v, out_v, gsem):
    wid = lax.axis_index("s") * NC + lax.axis_index("c")
    fid = wid // WORKERS_PER_FEATURE
    bag0 = (wid % WORKERS_PER_FEATURE) * BAGS_PER_WORKER

    def run_feature(ids_hbm, col):
        def fire(i, slot):
            base = (bag0 + i * CHUNK) * L
            pltpu.sync_copy(ids_hbm.at[pl.ds(base, CHUNK_IDS)], idx_v.at[slot])
            for j in range(GATHERS):
                pltpu.async_copy(
                    table_hbm.at[idx_v.at[slot].at[pl.ds(j * IDS_PER_GATHER, IDS_PER_GATHER)]],
                    rows_v.at[slot].at[pl.ds(j * IDS_PER_GATHER, IDS_PER_GATHER)],
                    gsem.at[slot],
                )

        def drain(slot):
            for j in range(GATHERS):
                pltpu.make_async_copy(
                    table_hbm.at[idx_v.at[slot].at[pl.ds(j * IDS_PER_GATHER, IDS_PER_GATHER)]],
                    rows_v.at[slot].at[pl.ds(j * IDS_PER_GATHER, IDS_PER_GATHER)],
                    gsem.at[slot],
                ).wait()

        def reduce_and_write(i, slot):
            rv = rows_v.at[slot]

            @plsc.parallel_loop(0, CHUNK, unroll=2)
            def _bag(b):
                r0 = b * L
                lo = rv[r0, pl.ds(0, 16)]
                hi = rv[r0, pl.ds(16, 16)]
                for k in range(1, L):
                    lo = lo + rv[r0 + k, pl.ds(0, 16)]
                    hi = hi + rv[r0 + k, pl.ds(16, 16)]
                out_v[b, pl.ds(0, 16)] = lo
                out_v[b, pl.ds(16, 16)] = hi

            pltpu.sync_copy(
                out_v, out_hbm.at[pl.ds(bag0 + i * CHUNK, CHUNK), pl.ds(col, D)]
            )

        fire(0, 0)

        def pair_body(g, carry):
            for b in range(NBUF):
                i = 2 * g + b

                @pl.when(i + 1 < CHUNKS)
                def _():
                    fire(i + 1, 1 - b)

                drain(b)
                reduce_and_write(i, b)
            return carry

        lax.fori_loop(0, CHUNKS // NBUF, pair_body, 0)

    @pl.when(fid == 0)
    def _():
        run_feature(v1_hbm, 0)

    @pl.when(fid == 1)
    def _():
        run_feature(v2_hbm, D)


def kernel(values_f1, lengths_f1, values_f2, lengths_f2, table):
    del lengths_f1, lengths_f2
    mesh = plsc.VectorSubcoreMesh(
        core_axis_name="c", subcore_axis_name="s", num_cores=NC, num_subcores=NS
    )
    run = pl.kernel(
        _sc_pool_kernel,
        out_type=jax.ShapeDtypeStruct((B, 2 * D), jnp.float32),
        mesh=mesh,
        scratch_types=[
            pltpu.VMEM((NBUF, CHUNK_IDS), jnp.int32),
            pltpu.VMEM((NBUF, CHUNK_IDS, D), jnp.float32),
            pltpu.VMEM((CHUNK, D), jnp.float32),
            pltpu.SemaphoreType.DMA((NBUF,)),
        ],
        compiler_params=pltpu.CompilerParams(use_tc_tiling_on_sc=False),
    )
    return run(values_f1, values_f2, table)


# speedup vs baseline: 15.3045x; 1.0563x over previous
import jax
import jax.numpy as jnp
from jax import lax
from jax.experimental import pallas as pl
from jax.experimental.pallas import tpu as pltpu
from jax.experimental.pallas import tpu_sc as plsc

B = 16384
L = 20
V = 1000000
D = 32

NC = 2
NS = 16
NW = NC * NS

COLS_FULL = (V // 128) * 128
TAIL = V - COLS_FULL
TCHUNK = 512
TOTAL_TCHUNKS = COLS_FULL // TCHUNK
TSTEPS = 62
TWORDS = TCHUNK * D

WORKERS_PER_FEATURE = NW // 2
BAGS_PER_WORKER = B // WORKERS_PER_FEATURE
CHUNK = 64
CHUNK_IDS = CHUNK * L
IDS_PER_GATHER = 128
GATHERS = CHUNK_IDS // IDS_PER_GATHER
CHUNKS = BAGS_PER_WORKER // CHUNK
NBUF = 2


def _sc_relayout_kernel(tt_hbm, tail_hbm, out_hbm, cbuf, tbuf, vtail, gsem, osem):
    wid = lax.axis_index("s") * NC + lax.axis_index("c")

    def tile_copies(k, slot):
        c0 = k * TCHUNK
        return [
            pltpu.make_async_copy(
                tt_hbm.at[pl.ds(8 * i, 8), pl.ds(c0 + 128 * jj, 128)],
                cbuf.at[pl.ds((slot * 16 + i * 4 + jj) * 8, 8), :],
                gsem.at[slot],
            )
            for i in range(4)
            for jj in range(4)
        ]

    def fire_in(t, slot):
        k = wid + 32 * t

        @pl.when(k < TOTAL_TCHUNKS)
        def _():
            for c in tile_copies(k, slot):
                c.start()

    def wait_in(t, slot):
        k = wid + 32 * t

        @pl.when(k < TOTAL_TCHUNKS)
        def _():
            for c in tile_copies(k, slot):
                c.wait()

    def wait_out(t, slot):
        k = wid + 32 * t

        @pl.when(jnp.logical_and(t >= 0, k < TOTAL_TCHUNKS))
        def _():
            pltpu.make_async_copy(
                tbuf.at[pl.ds(slot * TWORDS, TWORDS)],
                out_hbm.at[pl.ds(k * TWORDS, TWORDS)],
                osem.at[slot],
            ).wait()

    def transpose_fire_out(t, slot):
        k = wid + 32 * t

        @pl.when(k < TOTAL_TCHUNKS)
        def _():
            cb = cbuf.at[pl.ds(slot * 128, 128), :]
            tb = tbuf.at[pl.ds(slot * TWORDS, TWORDS)]
            iota = jnp.arange(16, dtype=jnp.int32)
            r_lo = (iota // 8) * 32 + (iota % 8)
            r_hi = r_lo + 64

            @plsc.parallel_loop(0, TCHUNK, unroll=4)
            def _w(w):
                jv = jnp.full((16,), (w // 128) * 8, dtype=jnp.int32)
                wv = jnp.full((16,), w % 128, dtype=jnp.int32)
                lo = plsc.load_gather(cb, [r_lo + jv, wv])
                hi = plsc.load_gather(cb, [r_hi + jv, wv])
                tb[pl.ds(w * D, 16)] = lo
                tb[pl.ds(w * D + 16, 16)] = hi

            pltpu.async_copy(
                tb, out_hbm.at[pl.ds(k * TWORDS, TWORDS)], osem.at[slot]
            )

    fire_in(0, 0)

    def pair_body(g, carry):
        for b2 in range(2):
            t = 2 * g + b2
            fire_in(t + 1, 1 - b2)
            wait_in(t, b2)
            wait_out(t - 2, b2)
            transpose_fire_out(t, b2)
        return carry

    lax.fori_loop(0, TSTEPS // 2, pair_body, 0)
    wait_out(TSTEPS - 2, 0)
    wait_out(TSTEPS - 1, 1)

    @pl.when(wid == 0)
    def _():
        pltpu.sync_copy(tail_hbm, vtail)
        pltpu.sync_copy(vtail, out_hbm.at[pl.ds(COLS_FULL * D, TAIL * D)])


def _sc_pool_kernel(v1_hbm, v2_hbm, table_hbm, out_hbm, idx_v, rows_v, out_v, gsem):
    wid = lax.axis_index("s") * NC + lax.axis_index("c")
    fid = wid // WORKERS_PER_FEATURE
    bag0 = (wid % WORKERS_PER_FEATURE) * BAGS_PER_WORKER

    def run_feature(ids_hbm, col):
        def fire(i, slot):
            base = (bag0 + i * CHUNK) * L
            pltpu.sync_copy(ids_hbm.at[pl.ds(base, CHUNK_IDS)], idx_v.at[slot])
            for j in range(GATHERS):
                pltpu.async_copy(
                    table_hbm.at[idx_v.at[slot].at[pl.ds(j * IDS_PER_GATHER, IDS_PER_GATHER)]],
                    rows_v.at[slot].at[pl.ds(j * IDS_PER_GATHER, IDS_PER_GATHER)],
                    gsem.at[slot],
                )

        def drain(slot):
            for j in range(GATHERS):
                pltpu.make_async_copy(
                    table_hbm.at[idx_v.at[slot].at[pl.ds(j * IDS_PER_GATHER, IDS_PER_GATHER)]],
                    rows_v.at[slot].at[pl.ds(j * IDS_PER_GATHER, IDS_PER_GATHER)],
                    gsem.at[slot],
                ).wait()

        def reduce_and_write(i, slot):
            rv = rows_v.at[slot]

            @plsc.parallel_loop(0, CHUNK, unroll=2)
            def _bag(b):
                r0 = b * L
                lo = rv[r0, pl.ds(0, 16)]
                hi = rv[r0, pl.ds(16, 16)]
                for k in range(1, L):
                    lo = lo + rv[r0 + k, pl.ds(0, 16)]
                    hi = hi + rv[r0 + k, pl.ds(16, 16)]
                out_v[b, pl.ds(0, 16)] = lo
                out_v[b, pl.ds(16, 16)] = hi

            pltpu.sync_copy(
                out_v, out_hbm.at[pl.ds(bag0 + i * CHUNK, CHUNK), pl.ds(col, D)]
            )

        fire(0, 0)

        def pair_body(g, carry):
            for b in range(NBUF):
                i = 2 * g + b

                @pl.when(i + 1 < CHUNKS)
                def _():
                    fire(i + 1, 1 - b)

                drain(b)
                reduce_and_write(i, b)
            return carry

        lax.fori_loop(0, CHUNKS // NBUF, pair_body, 0)

    @pl.when(fid == 0)
    def _():
        run_feature(v1_hbm, 0)

    @pl.when(fid == 1)
    def _():
        run_feature(v2_hbm, D)


def kernel(values_f1, lengths_f1, values_f2, lengths_f2, table):
    del lengths_f1, lengths_f2
    mesh = plsc.VectorSubcoreMesh(
        core_axis_name="c", subcore_axis_name="s", num_cores=NC, num_subcores=NS
    )

    tt = table.T
    tail = table[COLS_FULL:, :].reshape(TAIL * D)
    relayout = pl.kernel(
        _sc_relayout_kernel,
        out_type=jax.ShapeDtypeStruct((V * D,), jnp.float32),
        mesh=mesh,
        scratch_types=[
            pltpu.VMEM((NBUF * 128, 128), jnp.float32),
            pltpu.VMEM((NBUF * TWORDS,), jnp.float32),
            pltpu.VMEM((TAIL * D,), jnp.float32),
            pltpu.SemaphoreType.DMA((NBUF,)),
            pltpu.SemaphoreType.DMA((NBUF,)),
        ],
        compiler_params=pltpu.CompilerParams(
            use_tc_tiling_on_sc=True, needs_layout_passes=False
        ),
    )
    flat = relayout(tt, tail)
    table_rm = flat.reshape(V, D)

    pool = pl.kernel(
        _sc_pool_kernel,
        out_type=jax.ShapeDtypeStruct((B, 2 * D), jnp.float32),
        mesh=mesh,
        scratch_types=[
            pltpu.VMEM((NBUF, CHUNK_IDS), jnp.int32),
            pltpu.VMEM((NBUF, CHUNK_IDS, D), jnp.float32),
            pltpu.VMEM((CHUNK, D), jnp.float32),
            pltpu.SemaphoreType.DMA((NBUF,)),
        ],
        compiler_params=pltpu.CompilerParams(use_tc_tiling_on_sc=False),
    )
    return pool(values_f1, values_f2, table_rm)


# speedup vs baseline: 15.5702x; 1.0174x over previous
import jax
import jax.numpy as jnp
from jax import lax
from jax.experimental import pallas as pl
from jax.experimental.pallas import tpu as pltpu
from jax.experimental.pallas import tpu_sc as plsc

B = 16384
L = 20
V = 1000000
D = 32

NC = 2
NS = 16
NW = NC * NS

COLS_FULL = (V // 128) * 128
TAIL = V - COLS_FULL
TCHUNK = 512
TOTAL_TCHUNKS = COLS_FULL // TCHUNK
TSTEPS = 62
TWORDS = TCHUNK * D

WORKERS_PER_FEATURE = NW // 2
BAGS_PER_WORKER = B // WORKERS_PER_FEATURE
CHUNK = 64
CHUNK_IDS = CHUNK * L
IDS_PER_GATHER = 128
GATHERS = CHUNK_IDS // IDS_PER_GATHER
CHUNKS = BAGS_PER_WORKER // CHUNK
NBUF = 2


def _sc_relayout_kernel(tt_hbm, tail_hbm, out_hbm, cbuf, tbuf, vtail, gsem, osem):
    wid = lax.axis_index("s") * NC + lax.axis_index("c")

    def tile_copies(k, slot):
        c0 = k * TCHUNK
        return [
            pltpu.make_async_copy(
                tt_hbm.at[pl.ds(8 * i, 8), pl.ds(c0 + 128 * jj, 128)],
                cbuf.at[pl.ds((slot * 16 + i * 4 + jj) * 8, 8), :],
                gsem.at[slot],
            )
            for i in range(4)
            for jj in range(4)
        ]

    def fire_in(t, slot):
        k = wid + 32 * t

        @pl.when(k < TOTAL_TCHUNKS)
        def _():
            for c in tile_copies(k, slot):
                c.start()

    def wait_in(t, slot):
        k = wid + 32 * t

        @pl.when(k < TOTAL_TCHUNKS)
        def _():
            for c in tile_copies(k, slot):
                c.wait()

    def wait_out(t, slot):
        k = wid + 32 * t

        @pl.when(jnp.logical_and(t >= 0, k < TOTAL_TCHUNKS))
        def _():
            pltpu.make_async_copy(
                tbuf.at[pl.ds(slot * TWORDS, TWORDS)],
                out_hbm.at[pl.ds(k * TWORDS, TWORDS)],
                osem.at[slot],
            ).wait()

    def transpose_fire_out(t, slot):
        k = wid + 32 * t

        @pl.when(k < TOTAL_TCHUNKS)
        def _():
            cb = cbuf.at[pl.ds(slot * 128, 128), :]
            tb = tbuf.at[pl.ds(slot * TWORDS, TWORDS)]
            lane_base = jnp.arange(16, dtype=jnp.int32) * D

            @plsc.parallel_loop(0, 128, unroll=2)
            def _row(r):
                d = (r // 32) * 8 + (r % 8)
                jj = (r // 8) % 4
                base = lane_base + (d + jj * 128 * D)
                for w0 in range(0, 128, 16):
                    vals = cb[r, pl.ds(w0, 16)]
                    plsc.store_scatter(tb, [base + w0 * D], vals)

            pltpu.async_copy(
                tb, out_hbm.at[pl.ds(k * TWORDS, TWORDS)], osem.at[slot]
            )

    fire_in(0, 0)

    def pair_body(g, carry):
        for b2 in range(2):
            t = 2 * g + b2
            fire_in(t + 1, 1 - b2)
            wait_in(t, b2)
            wait_out(t - 2, b2)
            transpose_fire_out(t, b2)
        return carry

    lax.fori_loop(0, TSTEPS // 2, pair_body, 0)
    wait_out(TSTEPS - 2, 0)
    wait_out(TSTEPS - 1, 1)

    @pl.when(wid == 0)
    def _():
        pltpu.sync_copy(tail_hbm, vtail)
        pltpu.sync_copy(vtail, out_hbm.at[pl.ds(COLS_FULL * D, TAIL * D)])


def _sc_pool_kernel(v1_hbm, v2_hbm, table_hbm, out_hbm, idx_v, rows_v, out_v, gsem):
    wid = lax.axis_index("s") * NC + lax.axis_index("c")
    fid = wid // WORKERS_PER_FEATURE
    bag0 = (wid % WORKERS_PER_FEATURE) * BAGS_PER_WORKER

    def run_feature(ids_hbm, col):
        def fire(i, slot):
            base = (bag0 + i * CHUNK) * L
            pltpu.sync_copy(ids_hbm.at[pl.ds(base, CHUNK_IDS)], idx_v.at[slot])
            for j in range(GATHERS):
                pltpu.async_copy(
                    table_hbm.at[idx_v.at[slot].at[pl.ds(j * IDS_PER_GATHER, IDS_PER_GATHER)]],
                    rows_v.at[slot].at[pl.ds(j * IDS_PER_GATHER, IDS_PER_GATHER)],
                    gsem.at[slot],
                )

        def drain(slot):
            for j in range(GATHERS):
                pltpu.make_async_copy(
                    table_hbm.at[idx_v.at[slot].at[pl.ds(j * IDS_PER_GATHER, IDS_PER_GATHER)]],
                    rows_v.at[slot].at[pl.ds(j * IDS_PER_GATHER, IDS_PER_GATHER)],
                    gsem.at[slot],
                ).wait()

        def reduce_and_write(i, slot):
            rv = rows_v.at[slot]

            @plsc.parallel_loop(0, CHUNK, unroll=2)
            def _bag(b):
                r0 = b * L
                lo = rv[r0, pl.ds(0, 16)]
                hi = rv[r0, pl.ds(16, 16)]
                for k in range(1, L):
                    lo = lo + rv[r0 + k, pl.ds(0, 16)]
                    hi = hi + rv[r0 + k, pl.ds(16, 16)]
                out_v[b, pl.ds(0, 16)] = lo
                out_v[b, pl.ds(16, 16)] = hi

            pltpu.sync_copy(
                out_v, out_hbm.at[pl.ds(bag0 + i * CHUNK, CHUNK), pl.ds(col, D)]
            )

        fire(0, 0)

        def pair_body(g, carry):
            for b in range(NBUF):
                i = 2 * g + b

                @pl.when(i + 1 < CHUNKS)
                def _():
                    fire(i + 1, 1 - b)

                drain(b)
                reduce_and_write(i, b)
            return carry

        lax.fori_loop(0, CHUNKS // NBUF, pair_body, 0)

    @pl.when(fid == 0)
    def _():
        run_feature(v1_hbm, 0)

    @pl.when(fid == 1)
    def _():
        run_feature(v2_hbm, D)


def kernel(values_f1, lengths_f1, values_f2, lengths_f2, table):
    del lengths_f1, lengths_f2
    mesh = plsc.VectorSubcoreMesh(
        core_axis_name="c", subcore_axis_name="s", num_cores=NC, num_subcores=NS
    )

    tt = table.T
    tail = table[COLS_FULL:, :].reshape(TAIL * D)
    relayout = pl.kernel(
        _sc_relayout_kernel,
        out_type=jax.ShapeDtypeStruct((V * D,), jnp.float32),
        mesh=mesh,
        scratch_types=[
            pltpu.VMEM((NBUF * 128, 128), jnp.float32),
            pltpu.VMEM((NBUF * TWORDS,), jnp.float32),
            pltpu.VMEM((TAIL * D,), jnp.float32),
            pltpu.SemaphoreType.DMA((NBUF,)),
            pltpu.SemaphoreType.DMA((NBUF,)),
        ],
        compiler_params=pltpu.CompilerParams(
            use_tc_tiling_on_sc=True, needs_layout_passes=False
        ),
    )
    flat = relayout(tt, tail)
    table_rm = flat.reshape(V, D)

    pool = pl.kernel(
        _sc_pool_kernel,
        out_type=jax.ShapeDtypeStruct((B, 2 * D), jnp.float32),
        mesh=mesh,
        scratch_types=[
            pltpu.VMEM((NBUF, CHUNK_IDS), jnp.int32),
            pltpu.VMEM((NBUF, CHUNK_IDS, D), jnp.float32),
            pltpu.VMEM((CHUNK, D), jnp.float32),
            pltpu.SemaphoreType.DMA((NBUF,)),
        ],
        compiler_params=pltpu.CompilerParams(use_tc_tiling_on_sc=False),
    )
    return pool(values_f1, values_f2, table_rm)
